# baseline (speedup 1.0000x reference)
import jax
import jax.numpy as jnp
import numpy as np
from jax import lax
from jax.experimental import pallas as pl
from jax.experimental.pallas import tpu as pltpu
from jax.experimental.pallas import tpu_sc as plsc

NC, NS = 2, 16
NW = NC * NS
CH = 80
H = 128


def _gather_body(a_hbm, b_hbm, src_hbm, tgt_hbm, ga_hbm, gb_hbm,
                 idx_s, idx_t,
                 ra0, ra1, ra2, ra3, rb0, rb1, rb2, rb3,
                 sg0, sg1, sg2, sg3, sw0, sw1, sw2, sw3):
    nkc = idx_s.shape[0]
    e_per_w = nkc * CH
    c = lax.axis_index("c")
    s = lax.axis_index("s")
    w = s * NC + c
    base_w = w * e_per_w
    pltpu.sync_copy(src_hbm.at[w], idx_s)
    pltpu.sync_copy(tgt_hbm.at[w], idx_t)

    ras = [ra0, ra1, ra2, ra3]
    rbs = [rb0, rb1, rb2, rb3]
    sgs = [sg0, sg1, sg2, sg3]
    sws = [sw0, sw1, sw2, sw3]
    NB = 4

    def issue_gather(j, b):
        pltpu.async_copy(a_hbm.at[idx_s.at[j]], ras[b], sgs[b])
        pltpu.async_copy(b_hbm.at[idx_t.at[j]], rbs[b], sgs[b])

    def wait_gather(j, b):
        pltpu.make_async_copy(a_hbm.at[idx_s.at[j]], ras[b], sgs[b]).wait()
        pltpu.make_async_copy(b_hbm.at[idx_t.at[j]], rbs[b], sgs[b]).wait()

    def issue_wb(j, b):
        base = base_w + j * CH
        pltpu.async_copy(ras[b], ga_hbm.at[pl.ds(base, CH)], sws[b])
        pltpu.async_copy(rbs[b], gb_hbm.at[pl.ds(base, CH)], sws[b])

    def wait_wb(j, b):
        base = base_w + j * CH
        pltpu.make_async_copy(ras[b], ga_hbm.at[pl.ds(base, CH)], sws[b]).wait()
        pltpu.make_async_copy(rbs[b], gb_hbm.at[pl.ds(base, CH)], sws[b]).wait()

    issue_gather(0, 0)
    issue_gather(1, 1)

    n_outer = (nkc + NB - 1) // NB

    def outer(p, carry):
        for b in range(NB):
            j = p * NB + b
            jn = j + 2
            bn = (b + 2) % NB

            @pl.when(jnp.logical_and(jn >= NB, jn < nkc))
            def _():
                wait_wb(jn - NB, bn)

            @pl.when(jnp.logical_and(jn >= 2, jn < nkc))
            def _():
                issue_gather(jn, bn)

            @pl.when(j < nkc)
            def _():
                wait_gather(j, b)
                issue_wb(j, b)
        return carry

    lax.fori_loop(0, n_outer, outer, 0)
    for t in range(NB):
        j = nkc - NB + t
        if j >= 0:
            wait_wb(j, j % NB)


def _count_body(tgt_hbm, pcnt_hbm, idx_t, ones_v, sc0, scnt):
    npad = scnt.shape[0]
    nkc = idx_t.shape[0]
    rows_per_tile = npad // NS
    n_zchunk = rows_per_tile // CH
    c = lax.axis_index("c")
    s = lax.axis_index("s")
    w = s * NC + c

    def fill_ones(i, carry):
        def fill_col(k, cc):
            ones_v[i, pl.ds(k * 16, 16)] = jnp.zeros((16,), jnp.float32)
            return cc
        lax.fori_loop(0, H // 16, fill_col, 0)
        return carry

    lax.fori_loop(0, CH, fill_ones, 0)

    def zero_chunk(i, carry):
        r0 = s * rows_per_tile + i * CH
        pltpu.sync_copy(ones_v, scnt.at[pl.ds(r0, CH)])
        return carry

    lax.fori_loop(0, n_zchunk, zero_chunk, 0)

    def fill_ones2(i, carry):
        def fill_col(k, cc):
            ones_v[i, pl.ds(k * 16, 16)] = jnp.ones((16,), jnp.float32)
            return cc
        lax.fori_loop(0, H // 16, fill_col, 0)
        return carry

    lax.fori_loop(0, CH, fill_ones2, 0)
    plsc.subcore_barrier()

    pltpu.sync_copy(tgt_hbm.at[w], idx_t)

    K = 8
    n_outer = (nkc + K - 1) // K

    def outer(p, carry):
        for t in range(K):
            j = p * K + t

            @pl.when(j < nkc)
            def _():
                pltpu.async_copy(ones_v, scnt.at[idx_t.at[j]], sc0, add=True)
        for t in range(K):
            j = p * K + t

            @pl.when(j < nkc)
            def _():
                pltpu.make_async_copy(ones_v, scnt.at[idx_t.at[j]], sc0).wait()
        return carry

    lax.fori_loop(0, n_outer, outer, 0)
    plsc.subcore_barrier()

    def writeback(i, carry):
        r0 = s * rows_per_tile + i * CH
        pltpu.sync_copy(scnt.at[pl.ds(r0, CH)], ones_v)
        pltpu.sync_copy(ones_v, pcnt_hbm.at[c, pl.ds(r0, CH)])
        return carry

    lax.fori_loop(0, n_zchunk, writeback, 0)


def _scatter_body(m_hbm, tgt_hbm, psum_hbm, idx_t, rows, rows2,
                  sl0, sl1, ss0, ss1, ssum):
    npad = ssum.shape[0]
    nkc = idx_t.shape[0]
    e_per_w = nkc * CH
    rows_per_tile = npad // NS
    n_zchunk = rows_per_tile // CH
    c = lax.axis_index("c")
    s = lax.axis_index("s")
    w = s * NC + c

    def fill_rows(i, carry):
        def fill_col(k, cc):
            rows[i, pl.ds(k * 16, 16)] = jnp.zeros((16,), jnp.float32)
            return cc
        lax.fori_loop(0, H // 16, fill_col, 0)
        return carry

    lax.fori_loop(0, CH, fill_rows, 0)

    def zero_chunk(i, carry):
        r0 = s * rows_per_tile + i * CH
        pltpu.sync_copy(rows, ssum.at[pl.ds(r0, CH)])
        return carry

    lax.fori_loop(0, n_zchunk, zero_chunk, 0)
    plsc.subcore_barrier()

    pltpu.sync_copy(tgt_hbm.at[w], idx_t)
    rws = [rows, rows2]
    sls = [sl0, sl1]
    sss = [ss0, ss1]

    def issue_load(j, b):
        base = w * e_per_w + j * CH
        pltpu.async_copy(m_hbm.at[pl.ds(base, CH)], rws[b], sls[b])

    def wait_load(j, b):
        base = w * e_per_w + j * CH
        pltpu.make_async_copy(m_hbm.at[pl.ds(base, CH)], rws[b], sls[b]).wait()

    def issue_scatter(j, b):
        pltpu.async_copy(rws[b], ssum.at[idx_t.at[j]], sss[b], add=True)

    def wait_scatter(j, b):
        pltpu.make_async_copy(rws[b], ssum.at[idx_t.at[j]], sss[b]).wait()

    issue_load(0, 0)
    n_outer = (nkc + 1) // 2

    def outer(p, carry):
        for b in range(2):
            j = 2 * p + b
            jn = j + 1
            bn = (b + 1) % 2

            @pl.when(j < nkc)
            def _():
                wait_load(j, b)

            @pl.when(jnp.logical_and(jn >= 2, jn < nkc))
            def _():
                wait_scatter(jn - 2, bn)

            @pl.when(jn < nkc)
            def _():
                issue_load(jn, bn)

            @pl.when(j < nkc)
            def _():
                issue_scatter(j, b)
        return carry

    lax.fori_loop(0, n_outer, outer, 0)
    for t in range(2):
        j = nkc - 2 + t
        if j >= 0:
            wait_scatter(j, j % 2)
    plsc.subcore_barrier()

    def writeback(i, carry):
        r0 = s * rows_per_tile + i * CH
        pltpu.sync_copy(ssum.at[pl.ds(r0, CH)], rows)
        pltpu.sync_copy(rows, psum_hbm.at[c, pl.ds(r0, CH)])
        return carry

    lax.fori_loop(0, n_zchunk, writeback, 0)


def _ab_body(hv_ref, wsT_ref, wtT_ref, a_ref, b_ref):
    x = hv_ref[...]
    a_ref[...] = jnp.dot(x, wsT_ref[...], preferred_element_type=jnp.float32)
    b_ref[...] = jnp.dot(x, wtT_ref[...], preferred_element_type=jnp.float32)


def _mlp_body(he_ref, ga_ref, gb_ref, w1T_ref, b1_ref, w2T_ref, b2_ref,
              w3T_ref, b3_ref, m_ref):
    ga_i = ga_ref[...]
    gb_i = gb_ref[...]
    mask = jnp.int32(-65536)
    lo = (lax.bitcast_convert_type(ga_i << 16, jnp.float32)
          + lax.bitcast_convert_type(gb_i << 16, jnp.float32))
    hi = (lax.bitcast_convert_type(ga_i & mask, jnp.float32)
          + lax.bitcast_convert_type(gb_i & mask, jnp.float32))
    g = jnp.concatenate([lo, hi], axis=1)
    x = jnp.dot(he_ref[...], w1T_ref[...], preferred_element_type=jnp.float32)
    x = jnp.maximum(x + g + b1_ref[...], 0.0)
    x = jnp.dot(x, w2T_ref[...], preferred_element_type=jnp.float32)
    x = jnp.maximum(x + b2_ref[...], 0.0)
    x = jnp.dot(x, w3T_ref[...], preferred_element_type=jnp.float32)
    m_ref[...] = x + b3_ref[...]


def _node_body(ps_ref, pc_ref, hv_ref, d1T_ref, d1b_ref, d2T_ref, d2b_ref,
               g1_ref, bb1_ref, g2_ref, bb2_ref, out_ref):
    num = ps_ref[0] + ps_ref[1]
    cnt = pc_ref[0][:, 0:1] + pc_ref[1][:, 0:1]
    x = hv_ref[...] + num / cnt
    mu = jnp.mean(x, axis=-1, keepdims=True)
    var = jnp.mean((x - mu) ** 2, axis=-1, keepdims=True)
    x = (x - mu) / jnp.sqrt(var + 1e-5) * g1_ref[...] + bb1_ref[...]
    t = jnp.dot(x, d1T_ref[...], preferred_element_type=jnp.float32)
    t = jnp.maximum(t + d1b_ref[...], 0.0)
    y = jnp.dot(t, d2T_ref[...], preferred_element_type=jnp.float32)
    y = x + y + d2b_ref[...]
    mu = jnp.mean(y, axis=-1, keepdims=True)
    var = jnp.mean((y - mu) ** 2, axis=-1, keepdims=True)
    out_ref[...] = (y - mu) / jnp.sqrt(var + 1e-5) * g2_ref[...] + bb2_ref[...]


def kernel(h_V, h_E, edge_idx, W1_w, W1_b, W2_w, W2_b, W3_w, W3_b,
           d1_w, d1_b, d2_w, d2_b, ln1_g, ln1_b, ln2_g, ln2_b):
    n, h = h_V.shape
    e = h_E.shape[0]
    assert h == H and e % (NW * CH) == 0
    nkc = e // (NW * CH)
    bn = 2000
    be = 8000
    npad = 10240

    src_r = edge_idx[0].reshape(NW, nkc, CH)
    tgt_r = edge_idx[1].reshape(NW, nkc, CH)
    perm = np.concatenate([np.arange(0, H, 2), np.arange(1, H, 2)])
    w1eT = W1_w[:, :H].T[:, perm]
    w1sT = W1_w[:, H:2 * H].T
    w1tT = W1_w[:, 2 * H:].T
    b1_p = W1_b[perm]
    w2T_p = W2_w.T[perm, :]

    a_tab, b_tab = pl.pallas_call(
        _ab_body,
        grid=(n // bn,),
        in_specs=[
            pl.BlockSpec((bn, H), lambda i: (i, 0)),
            pl.BlockSpec((H, H), lambda i: (0, 0)),
            pl.BlockSpec((H, H), lambda i: (0, 0)),
        ],
        out_specs=[
            pl.BlockSpec((bn, H), lambda i: (i, 0)),
            pl.BlockSpec((bn, H), lambda i: (i, 0)),
        ],
        out_shape=[
            jax.ShapeDtypeStruct((n, H), jnp.float32),
            jax.ShapeDtypeStruct((n, H), jnp.float32),
        ],
    )(h_V, w1sT, w1tT)

    a_i = lax.bitcast_convert_type(
        a_tab.astype(jnp.bfloat16).reshape(n, H // 2, 2), jnp.int32)
    b_i = lax.bitcast_convert_type(
        b_tab.astype(jnp.bfloat16).reshape(n, H // 2, 2), jnp.int32)

    mesh = plsc.VectorSubcoreMesh(core_axis_name="c", subcore_axis_name="s")
    ga, gb = pl.kernel(
        _gather_body,
        out_type=(
            jax.ShapeDtypeStruct((e, H // 2), jnp.int32),
            jax.ShapeDtypeStruct((e, H // 2), jnp.int32),
        ),
        mesh=mesh,
        compiler_params=pltpu.CompilerParams(use_tc_tiling_on_sc=False),
        scratch_types=(
            [pltpu.VMEM((nkc, CH), jnp.int32)] * 2
            + [pltpu.VMEM((CH, H // 2), jnp.int32)] * 8
            + [pltpu.SemaphoreType.DMA] * 8
        ),
    )(a_i, b_i, src_r, tgt_r)

    pcnt, = pl.kernel(
        _count_body,
        out_type=(jax.ShapeDtypeStruct((NC, npad, H), jnp.float32),),
        mesh=mesh,
        scratch_types=[
            pltpu.VMEM((nkc, CH), jnp.int32),
            pltpu.VMEM((CH, H), jnp.float32),
            pltpu.SemaphoreType.DMA,
            pltpu.VMEM_SHARED((npad, H), jnp.float32),
        ],
    )(tgt_r)

    m = pl.pallas_call(
        _mlp_body,
        grid=(e // be,),
        in_specs=[
            pl.BlockSpec((be, H), lambda i: (i, 0)),
            pl.BlockSpec((be, H // 2), lambda i: (i, 0)),
            pl.BlockSpec((be, H // 2), lambda i: (i, 0)),
            pl.BlockSpec((H, H), lambda i: (0, 0)),
            pl.BlockSpec((1, H), lambda i: (0, 0)),
            pl.BlockSpec((H, H), lambda i: (0, 0)),
            pl.BlockSpec((1, H), lambda i: (0, 0)),
            pl.BlockSpec((H, H), lambda i: (0, 0)),
            pl.BlockSpec((1, H), lambda i: (0, 0)),
        ],
        out_specs=pl.BlockSpec((be, H), lambda i: (i, 0)),
        out_shape=jax.ShapeDtypeStruct((e, H), jnp.float32),
    )(h_E, ga, gb, w1eT, b1_p.reshape(1, H), w2T_p, W2_b.reshape(1, H),
      W3_w.T, W3_b.reshape(1, H))

    psum, = pl.kernel(
        _scatter_body,
        out_type=(jax.ShapeDtypeStruct((NC, npad, H), jnp.float32),),
        mesh=mesh,
        scratch_types=(
            [pltpu.VMEM((nkc, CH), jnp.int32)]
            + [pltpu.VMEM((CH, H), jnp.float32)] * 2
            + [pltpu.SemaphoreType.DMA] * 4
            + [pltpu.VMEM_SHARED((npad, H), jnp.float32)]
        ),
    )(m, tgt_r)

    out = pl.pallas_call(
        _node_body,
        grid=(n // bn,),
        in_specs=[
            pl.BlockSpec((NC, bn, H), lambda i: (0, i, 0)),
            pl.BlockSpec((NC, bn, H), lambda i: (0, i, 0)),
            pl.BlockSpec((bn, H), lambda i: (i, 0)),
            pl.BlockSpec((H, 4 * H), lambda i: (0, 0)),
            pl.BlockSpec((1, 4 * H), lambda i: (0, 0)),
            pl.BlockSpec((4 * H, H), lambda i: (0, 0)),
            pl.BlockSpec((1, H), lambda i: (0, 0)),
            pl.BlockSpec((1, H), lambda i: (0, 0)),
            pl.BlockSpec((1, H), lambda i: (0, 0)),
            pl.BlockSpec((1, H), lambda i: (0, 0)),
            pl.BlockSpec((1, H), lambda i: (0, 0)),
        ],
        out_specs=pl.BlockSpec((bn, H), lambda i: (i, 0)),
        out_shape=jax.ShapeDtypeStruct((n, H), jnp.float32),
    )(psum, pcnt, h_V, d1_w.T, d1_b.reshape(1, -1), d2_w.T,
      d2_b.reshape(1, H), ln1_g.reshape(1, H), ln1_b.reshape(1, H),
      ln2_g.reshape(1, H), ln2_b.reshape(1, H))
    return out


# speedup vs baseline: 1.4074x; 1.4074x over previous
import jax
import jax.numpy as jnp
import numpy as np
from jax import lax
from jax.experimental import pallas as pl
from jax.experimental.pallas import tpu as pltpu
from jax.experimental.pallas import tpu_sc as plsc

NC, NS = 2, 16
NW = NC * NS
CH = 80
H = 128


def _gather_body(a_hbm, b_hbm, src_hbm, tgt_hbm, ga_hbm, gb_hbm,
                 idx_s, idx_t,
                 ra0, ra1, ra2, ra3, rb0, rb1, rb2, rb3,
                 sg0, sg1, sg2, sg3, sw0, sw1, sw2, sw3):
    nkc = idx_s.shape[0]
    e_per_w = nkc * CH
    c = lax.axis_index("c")
    s = lax.axis_index("s")
    w = s * NC + c
    base_w = w * e_per_w
    pltpu.sync_copy(src_hbm.at[w], idx_s)
    pltpu.sync_copy(tgt_hbm.at[w], idx_t)

    ras = [ra0, ra1, ra2, ra3]
    rbs = [rb0, rb1, rb2, rb3]
    sgs = [sg0, sg1, sg2, sg3]
    sws = [sw0, sw1, sw2, sw3]
    NB = 4

    def issue_gather(j, b):
        pltpu.async_copy(a_hbm.at[idx_s.at[j]], ras[b], sgs[b])
        pltpu.async_copy(b_hbm.at[idx_t.at[j]], rbs[b], sgs[b])

    def wait_gather(j, b):
        pltpu.make_async_copy(a_hbm.at[idx_s.at[j]], ras[b], sgs[b]).wait()
        pltpu.make_async_copy(b_hbm.at[idx_t.at[j]], rbs[b], sgs[b]).wait()

    def issue_wb(j, b):
        base = base_w + j * CH
        pltpu.async_copy(ras[b], ga_hbm.at[pl.ds(base, CH)], sws[b])
        pltpu.async_copy(rbs[b], gb_hbm.at[pl.ds(base, CH)], sws[b])

    def wait_wb(j, b):
        base = base_w + j * CH
        pltpu.make_async_copy(ras[b], ga_hbm.at[pl.ds(base, CH)], sws[b]).wait()
        pltpu.make_async_copy(rbs[b], gb_hbm.at[pl.ds(base, CH)], sws[b]).wait()

    issue_gather(0, 0)
    issue_gather(1, 1)
    issue_gather(2, 2)

    n_outer = (nkc + NB - 1) // NB

    def outer(p, carry):
        for b in range(NB):
            j = p * NB + b
            jn = j + 3
            bn = (b + 3) % NB

            @pl.when(jnp.logical_and(jn >= NB, jn < nkc))
            def _():
                wait_wb(jn - NB, bn)

            @pl.when(jnp.logical_and(jn >= 3, jn < nkc))
            def _():
                issue_gather(jn, bn)

            @pl.when(j < nkc)
            def _():
                wait_gather(j, b)
                issue_wb(j, b)
        return carry

    lax.fori_loop(0, n_outer, outer, 0)
    for t in range(NB):
        j = nkc - NB + t
        if j >= 0:
            wait_wb(j, j % NB)


def _count_body(tgt_hbm, pcnt_hbm, idx_t, ones_v, sc0, scnt):
    npad = scnt.shape[0]
    nkc = idx_t.shape[0]
    rows_per_tile = npad // NS
    n_zchunk = rows_per_tile // CH
    c = lax.axis_index("c")
    s = lax.axis_index("s")
    w = s * NC + c

    def fill_ones(i, carry):
        def fill_col(k, cc):
            ones_v[i, pl.ds(k * 16, 16)] = jnp.zeros((16,), jnp.float32)
            return cc
        lax.fori_loop(0, H // 16, fill_col, 0)
        return carry

    lax.fori_loop(0, CH, fill_ones, 0)

    def zero_chunk(i, carry):
        r0 = s * rows_per_tile + i * CH
        pltpu.sync_copy(ones_v, scnt.at[pl.ds(r0, CH)])
        return carry

    lax.fori_loop(0, n_zchunk, zero_chunk, 0)

    def fill_ones2(i, carry):
        def fill_col(k, cc):
            ones_v[i, pl.ds(k * 16, 16)] = jnp.ones((16,), jnp.float32)
            return cc
        lax.fori_loop(0, H // 16, fill_col, 0)
        return carry

    lax.fori_loop(0, CH, fill_ones2, 0)
    plsc.subcore_barrier()

    pltpu.sync_copy(tgt_hbm.at[w], idx_t)

    K = 8
    n_outer = (nkc + K - 1) // K

    def outer(p, carry):
        for t in range(K):
            j = p * K + t

            @pl.when(j < nkc)
            def _():
                pltpu.async_copy(ones_v, scnt.at[idx_t.at[j]], sc0, add=True)
        for t in range(K):
            j = p * K + t

            @pl.when(j < nkc)
            def _():
                pltpu.make_async_copy(ones_v, scnt.at[idx_t.at[j]], sc0).wait()
        return carry

    lax.fori_loop(0, n_outer, outer, 0)
    plsc.subcore_barrier()

    def writeback(i, carry):
        r0 = s * rows_per_tile + i * CH
        pltpu.sync_copy(scnt.at[pl.ds(r0, CH)], ones_v)
        pltpu.sync_copy(ones_v, pcnt_hbm.at[c, pl.ds(r0, CH)])
        return carry

    lax.fori_loop(0, n_zchunk, writeback, 0)


def _scatter_body(m_hbm, tgt_hbm, psum_hbm, idx_t, rows, rows2, rows3,
                  sl0, sl1, sl2, ss0, ss1, ss2, ssum):
    npad = ssum.shape[0]
    nkc = idx_t.shape[0]
    e_per_w = nkc * CH
    rows_per_tile = npad // NS
    n_zchunk = rows_per_tile // CH
    c = lax.axis_index("c")
    s = lax.axis_index("s")
    w = s * NC + c

    def fill_rows(i, carry):
        def fill_col(k, cc):
            rows[i, pl.ds(k * 16, 16)] = jnp.zeros((16,), jnp.float32)
            return cc
        lax.fori_loop(0, H // 16, fill_col, 0)
        return carry

    lax.fori_loop(0, CH, fill_rows, 0)

    def zero_chunk(i, carry):
        r0 = s * rows_per_tile + i * CH
        pltpu.sync_copy(rows, ssum.at[pl.ds(r0, CH)])
        return carry

    lax.fori_loop(0, n_zchunk, zero_chunk, 0)
    plsc.subcore_barrier()

    pltpu.sync_copy(tgt_hbm.at[w], idx_t)
    rws = [rows, rows2, rows3]
    sls = [sl0, sl1, sl2]
    sss = [ss0, ss1, ss2]
    NBS = 3

    def issue_load(j, b):
        base = w * e_per_w + j * CH
        pltpu.async_copy(m_hbm.at[pl.ds(base, CH)], rws[b], sls[b])

    def wait_load(j, b):
        base = w * e_per_w + j * CH
        pltpu.make_async_copy(m_hbm.at[pl.ds(base, CH)], rws[b], sls[b]).wait()

    def issue_scatter(j, b):
        pltpu.async_copy(rws[b], ssum.at[idx_t.at[j]], sss[b], add=True)

    def wait_scatter(j, b):
        pltpu.make_async_copy(rws[b], ssum.at[idx_t.at[j]], sss[b]).wait()

    issue_load(0, 0)
    issue_load(1, 1)
    n_outer = (nkc + NBS - 1) // NBS

    def outer(p, carry):
        for b in range(NBS):
            j = NBS * p + b
            jn = j + 2
            bn = (b + 2) % NBS

            @pl.when(j < nkc)
            def _():
                wait_load(j, b)

            @pl.when(jnp.logical_and(jn >= NBS, jn < nkc))
            def _():
                wait_scatter(jn - NBS, bn)

            @pl.when(jnp.logical_and(jn >= 2, jn < nkc))
            def _():
                issue_load(jn, bn)

            @pl.when(j < nkc)
            def _():
                issue_scatter(j, b)
        return carry

    lax.fori_loop(0, n_outer, outer, 0)
    for t in range(NBS):
        j = nkc - NBS + t
        if j >= 0:
            wait_scatter(j, j % NBS)
    plsc.subcore_barrier()

    def writeback(i, carry):
        r0 = s * rows_per_tile + i * CH
        pltpu.sync_copy(ssum.at[pl.ds(r0, CH)], rows)
        pltpu.sync_copy(rows, psum_hbm.at[c, pl.ds(r0, CH)])
        return carry

    lax.fori_loop(0, n_zchunk, writeback, 0)


def _ab_body(hv_ref, wsT_ref, wtT_ref, a_ref, b_ref):
    x = hv_ref[...]
    a_ref[...] = jnp.dot(x, wsT_ref[...], preferred_element_type=jnp.float32)
    b_ref[...] = jnp.dot(x, wtT_ref[...], preferred_element_type=jnp.float32)


def _mlp_body(he_ref, ga_ref, gb_ref, w1T_ref, b1_ref, w2T_ref, b2_ref,
              w3T_ref, b3_ref, m_ref):
    x = jnp.dot(he_ref[...], w1T_ref[...], preferred_element_type=jnp.float32)
    x = jnp.maximum(x + ga_ref[...] + gb_ref[...] + b1_ref[...], 0.0)
    x = jnp.dot(x, w2T_ref[...], preferred_element_type=jnp.float32)
    x = jnp.maximum(x + b2_ref[...], 0.0)
    x = jnp.dot(x, w3T_ref[...], preferred_element_type=jnp.float32)
    m_ref[...] = x + b3_ref[...]


def _node_body(ps_ref, pc_ref, hv_ref, d1T_ref, d1b_ref, d2T_ref, d2b_ref,
               g1_ref, bb1_ref, g2_ref, bb2_ref, out_ref):
    num = ps_ref[0] + ps_ref[1]
    cnt = pc_ref[0][:, 0:1] + pc_ref[1][:, 0:1]
    x = hv_ref[...] + num / cnt
    mu = jnp.mean(x, axis=-1, keepdims=True)
    var = jnp.mean((x - mu) ** 2, axis=-1, keepdims=True)
    x = (x - mu) / jnp.sqrt(var + 1e-5) * g1_ref[...] + bb1_ref[...]
    t = jnp.dot(x, d1T_ref[...], preferred_element_type=jnp.float32)
    t = jnp.maximum(t + d1b_ref[...], 0.0)
    y = jnp.dot(t, d2T_ref[...], preferred_element_type=jnp.float32)
    y = x + y + d2b_ref[...]
    mu = jnp.mean(y, axis=-1, keepdims=True)
    var = jnp.mean((y - mu) ** 2, axis=-1, keepdims=True)
    out_ref[...] = (y - mu) / jnp.sqrt(var + 1e-5) * g2_ref[...] + bb2_ref[...]


def kernel(h_V, h_E, edge_idx, W1_w, W1_b, W2_w, W2_b, W3_w, W3_b,
           d1_w, d1_b, d2_w, d2_b, ln1_g, ln1_b, ln2_g, ln2_b):
    n, h = h_V.shape
    e = h_E.shape[0]
    assert h == H and e % (NW * CH) == 0
    nkc = e // (NW * CH)
    bn = 2000
    be = 4000
    npad = 10240

    src_r = edge_idx[0].reshape(NW, nkc, CH)
    tgt_r = edge_idx[1].reshape(NW, nkc, CH)
    w1eT = W1_w[:, :H].T
    w1sT = W1_w[:, H:2 * H].T
    w1tT = W1_w[:, 2 * H:].T

    a_tab, b_tab = pl.pallas_call(
        _ab_body,
        grid=(n // bn,),
        in_specs=[
            pl.BlockSpec((bn, H), lambda i: (i, 0)),
            pl.BlockSpec((H, H), lambda i: (0, 0)),
            pl.BlockSpec((H, H), lambda i: (0, 0)),
        ],
        out_specs=[
            pl.BlockSpec((bn, H), lambda i: (i, 0)),
            pl.BlockSpec((bn, H), lambda i: (i, 0)),
        ],
        out_shape=[
            jax.ShapeDtypeStruct((n, H), jnp.float32),
            jax.ShapeDtypeStruct((n, H), jnp.float32),
        ],
    )(h_V, w1sT, w1tT)

    mesh = plsc.VectorSubcoreMesh(core_axis_name="c", subcore_axis_name="s")
    ga, gb = pl.kernel(
        _gather_body,
        out_type=(
            jax.ShapeDtypeStruct((e, H), jnp.float32),
            jax.ShapeDtypeStruct((e, H), jnp.float32),
        ),
        mesh=mesh,
        scratch_types=(
            [pltpu.VMEM((nkc, CH), jnp.int32)] * 2
            + [pltpu.VMEM((CH, H), jnp.float32)] * 8
            + [pltpu.SemaphoreType.DMA] * 8
        ),
    )(a_tab, b_tab, src_r, tgt_r)

    pcnt, = pl.kernel(
        _count_body,
        out_type=(jax.ShapeDtypeStruct((NC, npad, H), jnp.float32),),
        mesh=mesh,
        scratch_types=[
            pltpu.VMEM((nkc, CH), jnp.int32),
            pltpu.VMEM((CH, H), jnp.float32),
            pltpu.SemaphoreType.DMA,
            pltpu.VMEM_SHARED((npad, H), jnp.float32),
        ],
    )(tgt_r)

    m = pl.pallas_call(
        _mlp_body,
        grid=(e // be,),
        in_specs=[
            pl.BlockSpec((be, H), lambda i: (i, 0)),
            pl.BlockSpec((be, H), lambda i: (i, 0)),
            pl.BlockSpec((be, H), lambda i: (i, 0)),
            pl.BlockSpec((H, H), lambda i: (0, 0)),
            pl.BlockSpec((1, H), lambda i: (0, 0)),
            pl.BlockSpec((H, H), lambda i: (0, 0)),
            pl.BlockSpec((1, H), lambda i: (0, 0)),
            pl.BlockSpec((H, H), lambda i: (0, 0)),
            pl.BlockSpec((1, H), lambda i: (0, 0)),
        ],
        out_specs=pl.BlockSpec((be, H), lambda i: (i, 0)),
        out_shape=jax.ShapeDtypeStruct((e, H), jnp.float32),
    )(h_E, ga, gb, w1eT, W1_b.reshape(1, H), W2_w.T, W2_b.reshape(1, H),
      W3_w.T, W3_b.reshape(1, H))

    psum, = pl.kernel(
        _scatter_body,
        out_type=(jax.ShapeDtypeStruct((NC, npad, H), jnp.float32),),
        mesh=mesh,
        scratch_types=(
            [pltpu.VMEM((nkc, CH), jnp.int32)]
            + [pltpu.VMEM((CH, H), jnp.float32)] * 3
            + [pltpu.SemaphoreType.DMA] * 6
            + [pltpu.VMEM_SHARED((npad, H), jnp.float32)]
        ),
    )(m, tgt_r)

    out = pl.pallas_call(
        _node_body,
        grid=(n // bn,),
        in_specs=[
            pl.BlockSpec((NC, bn, H), lambda i: (0, i, 0)),
            pl.BlockSpec((NC, bn, H), lambda i: (0, i, 0)),
            pl.BlockSpec((bn, H), lambda i: (i, 0)),
            pl.BlockSpec((H, 4 * H), lambda i: (0, 0)),
            pl.BlockSpec((1, 4 * H), lambda i: (0, 0)),
            pl.BlockSpec((4 * H, H), lambda i: (0, 0)),
            pl.BlockSpec((1, H), lambda i: (0, 0)),
            pl.BlockSpec((1, H), lambda i: (0, 0)),
            pl.BlockSpec((1, H), lambda i: (0, 0)),
            pl.BlockSpec((1, H), lambda i: (0, 0)),
            pl.BlockSpec((1, H), lambda i: (0, 0)),
        ],
        out_specs=pl.BlockSpec((bn, H), lambda i: (i, 0)),
        out_shape=jax.ShapeDtypeStruct((n, H), jnp.float32),
    )(psum, pcnt, h_V, d1_w.T, d1_b.reshape(1, -1), d2_w.T,
      d2_b.reshape(1, H), ln1_g.reshape(1, H), ln1_b.reshape(1, H),
      ln2_g.reshape(1, H), ln2_b.reshape(1, H))
    return out
